# baseline (speedup 1.0000x reference)
import functools

import jax
import jax.numpy as jnp
from jax import lax
from jax.experimental import pallas as pl
from jax.experimental.pallas import tpu as pltpu
from jax.experimental.pallas import tpu_sc as plsc

N_NODES = 10000
N_EDGES = 320000
D_FEAT = 128
D_EDGE = 4
MPL_OUT = 20
N_GRAPHS = 64

DP = 32
N_PAD = 10112
DUMMY_DST = 10008
NW = 32
E_PAD = 327680
EPW = E_PAD // NW
CHUNK = 1024
NCHUNK = EPW // CHUNK
SUBCH = CHUNK // 128
ROWS_PER_SUB = N_PAD // 16

E_ROWS = N_EDGES * D_EDGE // 128
E_ROWS_PAD = E_PAD * D_EDGE // 128
I_ROWS = N_EDGES // 128
I_ROWS_PAD = E_PAD // 128


def _prep_body(idx_ref, src_ref, dst_ref):
    src_ref[:I_ROWS] = idx_ref[0].reshape(I_ROWS, 128)
    src_ref[I_ROWS:] = jnp.zeros((I_ROWS_PAD - I_ROWS, 128), jnp.int32)
    dst_ref[:I_ROWS] = idx_ref[1].reshape(I_ROWS, 128)
    dst_ref[I_ROWS:] = jnp.full((I_ROWS_PAD - I_ROWS, 128), DUMMY_DST, jnp.int32)


def _node_proj_body(x_ref, w_ref, o_ref):
    o_ref[:N_NODES] = jnp.dot(x_ref[...], w_ref[...], preferred_element_type=jnp.float32)
    o_ref[N_NODES:] = jnp.zeros((N_PAD - N_NODES, DP), jnp.float32)


def _edge_sc_body(src_hbm, dst_hbm, nproj_hbm, eattr_hbm, we_hbm, zeros_hbm, out_hbm,
                  src_v, dst_v, rows_v, eattr_v, we_v, acc_sh,
                  sem_i0, sem_i1, sem_g0, sem_g1, sem_s0, sem_s1):
    cid = lax.axis_index("c")
    sid = lax.axis_index("s")
    wid = sid * 2 + cid

    pltpu.sync_copy(zeros_hbm.at[pl.ds(sid * ROWS_PER_SUB, ROWS_PER_SUB)],
                    acc_sh.at[pl.ds(sid * ROWS_PER_SUB, ROWS_PER_SUB)])
    pltpu.sync_copy(we_hbm, we_v)

    wk = [[we_v[k, pl.ds(h * 16, 16)] for h in range(DP // 16)] for k in range(D_EDGE)]
    bk = [we_v[D_EDGE, pl.ds(h * 16, 16)] for h in range(DP // 16)]

    plsc.subcore_barrier()

    sem_i = [sem_i0, sem_i1]
    sem_g = [sem_g0, sem_g1]
    sem_s = [sem_s0, sem_s1]

    def start_idx_dmas(g):
        b3 = g % 3
        base128 = wid * (EPW // 128) + g * SUBCH
        base_row = (wid * EPW + g * CHUNK) * D_EDGE // 128
        d = [pltpu.async_copy(src_hbm.at[pl.ds(base128, SUBCH)], src_v.at[b3], sem_i[g % 2]),
             pltpu.async_copy(dst_hbm.at[pl.ds(base128, SUBCH)], dst_v.at[b3], sem_i[g % 2]),
             pltpu.async_copy(eattr_hbm.at[pl.ds(base_row, CHUNK * D_EDGE // 128)],
                              eattr_v.at[b3], sem_i[g % 2])]
        return d

    def start_gathers(g):
        b, b3 = g % 2, g % 3
        return [pltpu.async_copy(nproj_hbm.at[src_v.at[b3].at[jj]],
                                 rows_v.at[b].at[pl.ds(jj * 128, 128)], sem_g[b])
                for jj in range(SUBCH)]

    def start_scatters(g):
        b, b3 = g % 2, g % 3
        return [pltpu.async_copy(rows_v.at[b].at[pl.ds(jj * 128, 128)],
                                 acc_sh.at[dst_v.at[b3].at[jj]], sem_s[b], add=True)
                for jj in range(SUBCH)]

    def compute(g):
        b, b3 = g % 2, g % 3

        def row_body(i, _):
            for jg in range(128 // 16):
                ev = eattr_v[b3, i, pl.ds(jg * 16, 16)]
                for t in range(4):
                    r = i * 32 + jg * 4 + t
                    e = [ev[t * D_EDGE + k] for k in range(D_EDGE)]
                    for h in range(DP // 16):
                        sl = pl.ds(h * 16, 16)
                        ep = bk[h]
                        for k in range(D_EDGE):
                            ep = ep + e[k] * wk[k][h]
                        rows_v[b, r, sl] = jnp.maximum(rows_v[b, r, sl] + ep, 0.0)
            return ()

        lax.fori_loop(0, CHUNK * D_EDGE // 128, row_body, ())

    idx_d = {0: start_idx_dmas(0)}
    for d in idx_d[0]:
        d.wait()
    gat_d = {0: start_gathers(0)}
    idx_d[1] = start_idx_dmas(1)
    sca_d = {}
    for g in range(NCHUNK):
        if g + 1 < NCHUNK:
            for d in idx_d[g + 1]:
                d.wait()
            if g >= 1:
                for d in sca_d[g - 1]:
                    d.wait()
            gat_d[g + 1] = start_gathers(g + 1)
        for d in gat_d[g]:
            d.wait()
        compute(g)
        sca_d[g] = start_scatters(g)
        if g + 2 < NCHUNK:
            idx_d[g + 2] = start_idx_dmas(g + 2)
    for d in sca_d[NCHUNK - 2]:
        d.wait()
    for d in sca_d[NCHUNK - 1]:
        d.wait()

    plsc.subcore_barrier()
    pltpu.sync_copy(acc_sh.at[pl.ds(sid * ROWS_PER_SUB, ROWS_PER_SUB)],
                    out_hbm.at[cid, pl.ds(sid * ROWS_PER_SUB, ROWS_PER_SUB)])


def _head_body(p_ref, batch_ref, w1_ref, b1_ref, w2_ref, b2_ref, w3_ref, b3_ref, o_ref):
    x1 = p_ref[0] + p_ref[1]
    x2 = jnp.maximum(
        jnp.dot(x1, w1_ref[...], preferred_element_type=jnp.float32) + b1_ref[...], 0.0)
    seg = jax.lax.broadcasted_iota(jnp.int32, (N_GRAPHS, N_PAD), 0)
    mask = (seg == batch_ref[...]).astype(jnp.float32)
    pooled = jnp.dot(mask, x2, preferred_element_type=jnp.float32)
    x3 = jnp.maximum(
        jnp.dot(pooled, w2_ref[...], preferred_element_type=jnp.float32) + b2_ref[...], 0.0)
    o_ref[...] = jnp.dot(x3, w3_ref[...], preferred_element_type=jnp.float32) + b3_ref[...]


def kernel(edge_index, node_attr, edge_attr, batch, W_mpl, b_mpl, W1, b1, W2, b2, W3, b3):
    f32 = jnp.float32

    Wn = jnp.zeros((D_FEAT, DP), f32).at[:, :MPL_OUT].set(W_mpl[:D_FEAT])
    Web = jnp.zeros((D_EDGE + 1, DP), f32)
    Web = Web.at[:D_EDGE, :MPL_OUT].set(W_mpl[D_FEAT:])
    Web = Web.at[D_EDGE, :MPL_OUT].set(b_mpl)

    batch_pad = jnp.full((1, N_PAD), N_GRAPHS, jnp.int32).at[0, :N_NODES].set(batch)
    W1p = jnp.zeros((DP, 128), f32).at[:MPL_OUT, :10].set(W1)
    b1p = jnp.zeros((1, 128), f32).at[0, :10].set(b1)
    W2p = jnp.zeros((128, 128), f32).at[:10, :10].set(W2)
    b2p = jnp.zeros((1, 128), f32).at[0, :10].set(b2)
    W3p = jnp.zeros((128, 128), f32).at[:10, :1].set(W3)
    b3p = jnp.zeros((1, 128), f32).at[0, :1].set(b3)
    zeros_acc = jnp.zeros((N_PAD, DP), f32)

    src2d, dst2d = pl.pallas_call(
        _prep_body,
        out_shape=(jax.ShapeDtypeStruct((I_ROWS_PAD, 128), jnp.int32),
                   jax.ShapeDtypeStruct((I_ROWS_PAD, 128), jnp.int32)),
    )(edge_index)

    node_proj = pl.pallas_call(
        _node_proj_body,
        out_shape=jax.ShapeDtypeStruct((N_PAD, DP), f32),
    )(node_attr, Wn)

    mesh = plsc.VectorSubcoreMesh(core_axis_name="c", subcore_axis_name="s")
    edge_stage = functools.partial(
        pl.kernel,
        out_type=jax.ShapeDtypeStruct((2, N_PAD, DP), f32),
        mesh=mesh,
        scratch_types=[
            pltpu.VMEM((3, SUBCH, 128), jnp.int32),
            pltpu.VMEM((3, SUBCH, 128), jnp.int32),
            pltpu.VMEM((2, CHUNK, DP), f32),
            pltpu.VMEM((3, CHUNK * D_EDGE // 128, 128), f32),
            pltpu.VMEM((D_EDGE + 1, DP), f32),
            pltpu.VMEM_SHARED((N_PAD, DP), f32),
            pltpu.SemaphoreType.DMA,
            pltpu.SemaphoreType.DMA,
            pltpu.SemaphoreType.DMA,
            pltpu.SemaphoreType.DMA,
            pltpu.SemaphoreType.DMA,
            pltpu.SemaphoreType.DMA,
        ],
        compiler_params=pltpu.CompilerParams(use_tc_tiling_on_sc=False,
                                             needs_layout_passes=False),
    )(_edge_sc_body)
    eattr2d = jnp.zeros((E_ROWS_PAD, 128), f32).at[:E_ROWS].set(
        edge_attr.reshape(E_ROWS, 128))
    partials = edge_stage(src2d, dst2d, node_proj, eattr2d, Web, zeros_acc)

    out = pl.pallas_call(
        _head_body,
        out_shape=jax.ShapeDtypeStruct((N_GRAPHS, 128), f32),
    )(partials, batch_pad, W1p, b1p, W2p, b2p, W3p, b3p)
    return out[:, :1]


# speedup vs baseline: 2.2675x; 2.2675x over previous
import functools

import jax
import jax.numpy as jnp
from jax import lax
from jax.experimental import pallas as pl
from jax.experimental.pallas import tpu as pltpu
from jax.experimental.pallas import tpu_sc as plsc

N_NODES = 10000
N_EDGES = 320000
D_FEAT = 128
D_EDGE = 4
MPL_OUT = 20
N_GRAPHS = 64

DP = 32
N_PAD = 10112
DUMMY_DST = 10008
NW = 32
E_PAD = 327680
EPW = E_PAD // NW
CHUNK = 1024
NCHUNK = EPW // CHUNK
SUBCH = CHUNK // 128
ROWS_PER_SUB = N_PAD // 16

E_ROWS = N_EDGES * D_EDGE // 128
E_ROWS_PAD = E_PAD * D_EDGE // 128
I_ROWS = N_EDGES // 128
I_ROWS_PAD = E_PAD // 128


def _prep_body(idx_ref, src_ref, dst_ref):
    src_ref[:I_ROWS] = idx_ref[0].reshape(I_ROWS, 128)
    src_ref[I_ROWS:] = jnp.zeros((I_ROWS_PAD - I_ROWS, 128), jnp.int32)
    dst_ref[:I_ROWS] = idx_ref[1].reshape(I_ROWS, 128)
    dst_ref[I_ROWS:] = jnp.full((I_ROWS_PAD - I_ROWS, 128), DUMMY_DST, jnp.int32)


def _node_proj_body(x_ref, w_ref, o_ref):
    o_ref[:N_NODES] = jnp.dot(x_ref[...], w_ref[...], preferred_element_type=jnp.float32)
    o_ref[N_NODES:] = jnp.zeros((N_PAD - N_NODES, DP), jnp.float32)


def _edge_sc_body(src_hbm, dst_hbm, nproj_hbm, eattr_hbm, we_hbm, zeros_hbm, out_hbm,
                  src_v, dst_v, rows_v, eattr_v, we_v, acc_sh,
                  sem_i0, sem_i1, sem_g0, sem_g1, sem_s0, sem_s1):
    cid = lax.axis_index("c")
    sid = lax.axis_index("s")
    wid = sid * 2 + cid

    pltpu.sync_copy(zeros_hbm.at[pl.ds(sid * ROWS_PER_SUB, ROWS_PER_SUB)],
                    acc_sh.at[pl.ds(sid * ROWS_PER_SUB, ROWS_PER_SUB)])
    pltpu.sync_copy(we_hbm, we_v)

    wk = [[we_v[k, pl.ds(h * 16, 16)] for h in range(DP // 16)] for k in range(D_EDGE)]
    bk = [we_v[D_EDGE, pl.ds(h * 16, 16)] for h in range(DP // 16)]

    plsc.subcore_barrier()

    sem_i = [sem_i0, sem_i1]
    sem_g = [sem_g0, sem_g1]
    sem_s = [sem_s0, sem_s1]

    def start_idx_dmas(g):
        b3 = g % 3
        base128 = wid * (EPW // 128) + g * SUBCH
        base_e = wid * EPW + g * CHUNK
        d = [pltpu.async_copy(src_hbm.at[pl.ds(base128, SUBCH)], src_v.at[b3], sem_i[g % 2]),
             pltpu.async_copy(dst_hbm.at[pl.ds(base128, SUBCH)], dst_v.at[b3], sem_i[g % 2])]
        d += [pltpu.async_copy(eattr_hbm.at[k, pl.ds(base_e, CHUNK)],
                               eattr_v.at[b3, k], sem_i[g % 2])
              for k in range(D_EDGE)]
        return d

    def start_gathers(g):
        b, b3 = g % 2, g % 3
        return [pltpu.async_copy(nproj_hbm.at[src_v.at[b3].at[jj]],
                                 rows_v.at[b].at[pl.ds(jj * 128, 128)], sem_g[b])
                for jj in range(SUBCH)]

    def start_scatters(g):
        b, b3 = g % 2, g % 3
        return [pltpu.async_copy(rows_v.at[b].at[pl.ds(jj * 128, 128)],
                                 acc_sh.at[dst_v.at[b3].at[jj]], sem_s[b], add=True)
                for jj in range(SUBCH)]

    def compute(g):
        b, b3 = g % 2, g % 3

        def grp_body(m, _):
            ekv = [eattr_v[b3, k, pl.ds(m * 16, 16)] for k in range(D_EDGE)]
            for t in range(16):
                r = m * 16 + t
                for h in range(DP // 16):
                    sl = pl.ds(h * 16, 16)
                    ep = bk[h]
                    for k in range(D_EDGE):
                        ep = ep + ekv[k][t] * wk[k][h]
                    rows_v[b, r, sl] = jnp.maximum(rows_v[b, r, sl] + ep, 0.0)
            return ()

        lax.fori_loop(0, CHUNK // 16, grp_body, ())

    idx_d = {0: start_idx_dmas(0)}
    for d in idx_d[0]:
        d.wait()
    gat_d = {0: start_gathers(0)}
    idx_d[1] = start_idx_dmas(1)
    sca_d = {}
    for g in range(NCHUNK):
        if g + 1 < NCHUNK:
            for d in idx_d[g + 1]:
                d.wait()
            if g >= 1:
                for d in sca_d[g - 1]:
                    d.wait()
            gat_d[g + 1] = start_gathers(g + 1)
        for d in gat_d[g]:
            d.wait()
        compute(g)
        sca_d[g] = start_scatters(g)
        if g + 2 < NCHUNK:
            idx_d[g + 2] = start_idx_dmas(g + 2)
    for d in sca_d[NCHUNK - 2]:
        d.wait()
    for d in sca_d[NCHUNK - 1]:
        d.wait()

    plsc.subcore_barrier()
    pltpu.sync_copy(acc_sh.at[pl.ds(sid * ROWS_PER_SUB, ROWS_PER_SUB)],
                    out_hbm.at[cid, pl.ds(sid * ROWS_PER_SUB, ROWS_PER_SUB)])


def _head_body(p_ref, batch_ref, w1_ref, b1_ref, w2_ref, b2_ref, w3_ref, b3_ref, o_ref):
    x1 = p_ref[0] + p_ref[1]
    x2 = jnp.maximum(
        jnp.dot(x1, w1_ref[...], preferred_element_type=jnp.float32) + b1_ref[...], 0.0)
    seg = jax.lax.broadcasted_iota(jnp.int32, (N_GRAPHS, N_PAD), 0)
    mask = (seg == batch_ref[...]).astype(jnp.float32)
    pooled = jnp.dot(mask, x2, preferred_element_type=jnp.float32)
    x3 = jnp.maximum(
        jnp.dot(pooled, w2_ref[...], preferred_element_type=jnp.float32) + b2_ref[...], 0.0)
    o_ref[...] = jnp.dot(x3, w3_ref[...], preferred_element_type=jnp.float32) + b3_ref[...]


def kernel(edge_index, node_attr, edge_attr, batch, W_mpl, b_mpl, W1, b1, W2, b2, W3, b3):
    f32 = jnp.float32

    Wn = jnp.zeros((D_FEAT, DP), f32).at[:, :MPL_OUT].set(W_mpl[:D_FEAT])
    Web = jnp.zeros((D_EDGE + 1, DP), f32)
    Web = Web.at[:D_EDGE, :MPL_OUT].set(W_mpl[D_FEAT:])
    Web = Web.at[D_EDGE, :MPL_OUT].set(b_mpl)

    batch_pad = jnp.full((1, N_PAD), N_GRAPHS, jnp.int32).at[0, :N_NODES].set(batch)
    W1p = jnp.zeros((DP, 128), f32).at[:MPL_OUT, :10].set(W1)
    b1p = jnp.zeros((1, 128), f32).at[0, :10].set(b1)
    W2p = jnp.zeros((128, 128), f32).at[:10, :10].set(W2)
    b2p = jnp.zeros((1, 128), f32).at[0, :10].set(b2)
    W3p = jnp.zeros((128, 128), f32).at[:10, :1].set(W3)
    b3p = jnp.zeros((1, 128), f32).at[0, :1].set(b3)
    zeros_acc = jnp.zeros((N_PAD, DP), f32)

    src2d, dst2d = pl.pallas_call(
        _prep_body,
        out_shape=(jax.ShapeDtypeStruct((I_ROWS_PAD, 128), jnp.int32),
                   jax.ShapeDtypeStruct((I_ROWS_PAD, 128), jnp.int32)),
    )(edge_index)

    node_proj = pl.pallas_call(
        _node_proj_body,
        out_shape=jax.ShapeDtypeStruct((N_PAD, DP), f32),
    )(node_attr, Wn)

    mesh = plsc.VectorSubcoreMesh(core_axis_name="c", subcore_axis_name="s")
    edge_stage = functools.partial(
        pl.kernel,
        out_type=jax.ShapeDtypeStruct((2, N_PAD, DP), f32),
        mesh=mesh,
        scratch_types=[
            pltpu.VMEM((3, SUBCH, 128), jnp.int32),
            pltpu.VMEM((3, SUBCH, 128), jnp.int32),
            pltpu.VMEM((2, CHUNK, DP), f32),
            pltpu.VMEM((3, D_EDGE, CHUNK), f32),
            pltpu.VMEM((D_EDGE + 1, DP), f32),
            pltpu.VMEM_SHARED((N_PAD, DP), f32),
            pltpu.SemaphoreType.DMA,
            pltpu.SemaphoreType.DMA,
            pltpu.SemaphoreType.DMA,
            pltpu.SemaphoreType.DMA,
            pltpu.SemaphoreType.DMA,
            pltpu.SemaphoreType.DMA,
        ],
        compiler_params=pltpu.CompilerParams(use_tc_tiling_on_sc=False,
                                             needs_layout_passes=False),
    )(_edge_sc_body)
    eattrT = jnp.zeros((D_EDGE, E_PAD), f32).at[:, :N_EDGES].set(edge_attr.T)
    partials = edge_stage(src2d, dst2d, node_proj, eattrT, Web, zeros_acc)

    out = pl.pallas_call(
        _head_body,
        out_shape=jax.ShapeDtypeStruct((N_GRAPHS, 128), f32),
    )(partials, batch_pad, W1p, b1p, W2p, b2p, W3p, b3p)
    return out[:, :1]


# speedup vs baseline: 2.4680x; 1.0884x over previous
import functools

import jax
import jax.numpy as jnp
from jax import lax
from jax.experimental import pallas as pl
from jax.experimental.pallas import tpu as pltpu
from jax.experimental.pallas import tpu_sc as plsc

N_NODES = 10000
N_EDGES = 320000
D_FEAT = 128
D_EDGE = 4
MPL_OUT = 20
N_GRAPHS = 64

DP = 32
N_PAD = 10112
DUMMY_DST = 10008
NW = 32
E_PAD = 327680
EPW = E_PAD // NW
CHUNK = 1024
NCHUNK = EPW // CHUNK
SUBCH = CHUNK // 128
ROWS_PER_SUB = N_PAD // 16

E_ROWS = N_EDGES * D_EDGE // 128
E_ROWS_PAD = E_PAD * D_EDGE // 128
I_ROWS = N_EDGES // 128
I_ROWS_PAD = E_PAD // 128


def _prep_body(idx_ref, src_ref, dst_ref):
    src_ref[:I_ROWS] = idx_ref[0].reshape(I_ROWS, 128)
    src_ref[I_ROWS:] = jnp.zeros((I_ROWS_PAD - I_ROWS, 128), jnp.int32)
    dst_ref[:I_ROWS] = idx_ref[1].reshape(I_ROWS, 128)
    dst_ref[I_ROWS:] = jnp.full((I_ROWS_PAD - I_ROWS, 128), DUMMY_DST, jnp.int32)


def _node_proj_body(x_ref, w_ref, o_ref):
    o_ref[:N_NODES] = jnp.dot(x_ref[...], w_ref[...], preferred_element_type=jnp.float32)
    o_ref[N_NODES:] = jnp.zeros((N_PAD - N_NODES, DP), jnp.float32)


def _edge_sc_body(src_hbm, dst_hbm, nproj_hbm, eattr_hbm, we_hbm, zeros_hbm, out_hbm,
                  src_v, dst_v, rows_v, eattr_v, we_v, acc_sh, table_sh,
                  sem_i0, sem_i1, sem_g0, sem_g1, sem_s0, sem_s1):
    cid = lax.axis_index("c")
    sid = lax.axis_index("s")
    wid = sid * 2 + cid

    pltpu.sync_copy(zeros_hbm.at[pl.ds(sid * ROWS_PER_SUB, ROWS_PER_SUB)],
                    acc_sh.at[pl.ds(sid * ROWS_PER_SUB, ROWS_PER_SUB)])
    pltpu.sync_copy(nproj_hbm.at[pl.ds(sid * ROWS_PER_SUB, ROWS_PER_SUB)],
                    table_sh.at[pl.ds(sid * ROWS_PER_SUB, ROWS_PER_SUB)])
    pltpu.sync_copy(we_hbm, we_v)

    wk = [[we_v[k, pl.ds(h * 16, 16)] for h in range(DP // 16)] for k in range(D_EDGE)]
    bk = [we_v[D_EDGE, pl.ds(h * 16, 16)] for h in range(DP // 16)]

    plsc.subcore_barrier()

    sem_i = [sem_i0, sem_i1]
    sem_g = [sem_g0, sem_g1]
    sem_s = [sem_s0, sem_s1]

    def start_idx_dmas(g):
        b3 = g % 3
        base128 = wid * (EPW // 128) + g * SUBCH
        base_e = wid * EPW + g * CHUNK
        d = [pltpu.async_copy(src_hbm.at[pl.ds(base128, SUBCH)], src_v.at[b3], sem_i[g % 2]),
             pltpu.async_copy(dst_hbm.at[pl.ds(base128, SUBCH)], dst_v.at[b3], sem_i[g % 2])]
        d += [pltpu.async_copy(eattr_hbm.at[k, pl.ds(base_e, CHUNK)],
                               eattr_v.at[b3, k], sem_i[g % 2])
              for k in range(D_EDGE)]
        return d

    def start_gathers(g):
        b, b3 = g % 2, g % 3
        return [pltpu.async_copy(table_sh.at[src_v.at[b3].at[jj]],
                                 rows_v.at[b].at[pl.ds(jj * 128, 128)], sem_g[b])
                for jj in range(SUBCH)]

    def start_scatters(g):
        b, b3 = g % 2, g % 3
        return [pltpu.async_copy(rows_v.at[b].at[pl.ds(jj * 128, 128)],
                                 acc_sh.at[dst_v.at[b3].at[jj]], sem_s[b], add=True)
                for jj in range(SUBCH)]

    def compute(g):
        b, b3 = g % 2, g % 3

        def grp_body(m, _):
            ekv = [eattr_v[b3, k, pl.ds(m * 16, 16)] for k in range(D_EDGE)]
            for t in range(16):
                r = m * 16 + t
                for h in range(DP // 16):
                    sl = pl.ds(h * 16, 16)
                    ep = bk[h]
                    for k in range(D_EDGE):
                        ep = ep + ekv[k][t] * wk[k][h]
                    rows_v[b, r, sl] = jnp.maximum(rows_v[b, r, sl] + ep, 0.0)
            return ()

        lax.fori_loop(0, CHUNK // 16, grp_body, ())

    idx_d = {0: start_idx_dmas(0)}
    for d in idx_d[0]:
        d.wait()
    gat_d = {0: start_gathers(0)}
    idx_d[1] = start_idx_dmas(1)
    sca_d = {}
    for g in range(NCHUNK):
        if g + 1 < NCHUNK:
            for d in idx_d[g + 1]:
                d.wait()
            if g >= 1:
                for d in sca_d[g - 1]:
                    d.wait()
            gat_d[g + 1] = start_gathers(g + 1)
        for d in gat_d[g]:
            d.wait()
        compute(g)
        sca_d[g] = start_scatters(g)
        if g + 2 < NCHUNK:
            idx_d[g + 2] = start_idx_dmas(g + 2)
    for d in sca_d[NCHUNK - 2]:
        d.wait()
    for d in sca_d[NCHUNK - 1]:
        d.wait()

    plsc.subcore_barrier()
    pltpu.sync_copy(acc_sh.at[pl.ds(sid * ROWS_PER_SUB, ROWS_PER_SUB)],
                    out_hbm.at[cid, pl.ds(sid * ROWS_PER_SUB, ROWS_PER_SUB)])


def _head_body(p_ref, batch_ref, w1_ref, b1_ref, w2_ref, b2_ref, w3_ref, b3_ref, o_ref):
    x1 = p_ref[0] + p_ref[1]
    x2 = jnp.maximum(
        jnp.dot(x1, w1_ref[...], preferred_element_type=jnp.float32) + b1_ref[...], 0.0)
    seg = jax.lax.broadcasted_iota(jnp.int32, (N_GRAPHS, N_PAD), 0)
    mask = (seg == batch_ref[...]).astype(jnp.float32)
    pooled = jnp.dot(mask, x2, preferred_element_type=jnp.float32)
    x3 = jnp.maximum(
        jnp.dot(pooled, w2_ref[...], preferred_element_type=jnp.float32) + b2_ref[...], 0.0)
    o_ref[...] = jnp.dot(x3, w3_ref[...], preferred_element_type=jnp.float32) + b3_ref[...]


def kernel(edge_index, node_attr, edge_attr, batch, W_mpl, b_mpl, W1, b1, W2, b2, W3, b3):
    f32 = jnp.float32

    Wn = jnp.zeros((D_FEAT, DP), f32).at[:, :MPL_OUT].set(W_mpl[:D_FEAT])
    Web = jnp.zeros((D_EDGE + 1, DP), f32)
    Web = Web.at[:D_EDGE, :MPL_OUT].set(W_mpl[D_FEAT:])
    Web = Web.at[D_EDGE, :MPL_OUT].set(b_mpl)

    batch_pad = jnp.full((1, N_PAD), N_GRAPHS, jnp.int32).at[0, :N_NODES].set(batch)
    W1p = jnp.zeros((DP, 128), f32).at[:MPL_OUT, :10].set(W1)
    b1p = jnp.zeros((1, 128), f32).at[0, :10].set(b1)
    W2p = jnp.zeros((128, 128), f32).at[:10, :10].set(W2)
    b2p = jnp.zeros((1, 128), f32).at[0, :10].set(b2)
    W3p = jnp.zeros((128, 128), f32).at[:10, :1].set(W3)
    b3p = jnp.zeros((1, 128), f32).at[0, :1].set(b3)
    zeros_acc = jnp.zeros((N_PAD, DP), f32)

    src2d, dst2d = pl.pallas_call(
        _prep_body,
        out_shape=(jax.ShapeDtypeStruct((I_ROWS_PAD, 128), jnp.int32),
                   jax.ShapeDtypeStruct((I_ROWS_PAD, 128), jnp.int32)),
    )(edge_index)

    node_proj = pl.pallas_call(
        _node_proj_body,
        out_shape=jax.ShapeDtypeStruct((N_PAD, DP), f32),
    )(node_attr, Wn)

    mesh = plsc.VectorSubcoreMesh(core_axis_name="c", subcore_axis_name="s")
    edge_stage = functools.partial(
        pl.kernel,
        out_type=jax.ShapeDtypeStruct((2, N_PAD, DP), f32),
        mesh=mesh,
        scratch_types=[
            pltpu.VMEM((3, SUBCH, 128), jnp.int32),
            pltpu.VMEM((3, SUBCH, 128), jnp.int32),
            pltpu.VMEM((2, CHUNK, DP), f32),
            pltpu.VMEM((3, D_EDGE, CHUNK), f32),
            pltpu.VMEM((D_EDGE + 1, DP), f32),
            pltpu.VMEM_SHARED((N_PAD, DP), f32),
            pltpu.VMEM_SHARED((N_PAD, DP), f32),
            pltpu.SemaphoreType.DMA,
            pltpu.SemaphoreType.DMA,
            pltpu.SemaphoreType.DMA,
            pltpu.SemaphoreType.DMA,
            pltpu.SemaphoreType.DMA,
            pltpu.SemaphoreType.DMA,
        ],
        compiler_params=pltpu.CompilerParams(use_tc_tiling_on_sc=False,
                                             needs_layout_passes=False),
    )(_edge_sc_body)
    eattrT = jnp.zeros((D_EDGE, E_PAD), f32).at[:, :N_EDGES].set(edge_attr.T)
    partials = edge_stage(src2d, dst2d, node_proj, eattrT, Web, zeros_acc)

    out = pl.pallas_call(
        _head_body,
        out_shape=jax.ShapeDtypeStruct((N_GRAPHS, 128), f32),
    )(partials, batch_pad, W1p, b1p, W2p, b2p, W3p, b3p)
    return out[:, :1]
